# Initial kernel scaffold; baseline (speedup 1.0000x reference)
import jax
import jax.numpy as jnp
from jax.experimental import pallas as pl


def kernel(x, edge_index, W_in, b_in, eps1, W1a, b1a, W1b, b1b, eps2, W2a, b2a, W2b, b2b, eps3, W3a, b3a, W3b, b3b, Wc1, bc1, Wc2, bc2):
    raise NotImplementedError("write your pallas kernel here")


# speedup vs baseline: 6.5907x; 6.5907x over previous
import functools
import math

import jax
import jax.numpy as jnp
from jax import lax
from jax.experimental import pallas as pl
from jax.experimental.pallas import tpu as pltpu
from jax.experimental.pallas import tpu_sc as plsc

N = 10000
F = 128
H = 64
C = 2
E = 320000

NW = 32
CHUNK = 128
NCH = -(-(E // NW) // CHUNK)
EPW = NCH * CHUNK
EPAD = EPW * NW
NPAD = 10240
TROWS = NPAD // 16
ZCH = TROWS // CHUNK
INV_BN = float(1.0 / math.sqrt(1.0 + 1e-5))

RB = 1000


def _sc_agg(h, src, dst, zeros_tab):
    mesh = plsc.VectorSubcoreMesh(core_axis_name="c", subcore_axis_name="s")

    @functools.partial(
        pl.kernel,
        mesh=mesh,
        out_type=jax.ShapeDtypeStruct((2, NPAD, H), jnp.float32),
        scratch_types=[
            pltpu.VMEM((NCH, CHUNK), jnp.int32),
            pltpu.VMEM((NCH, CHUNK), jnp.int32),
            pltpu.VMEM((CHUNK, H), jnp.float32),
            pltpu.VMEM_SHARED((NPAD, H), jnp.float32),
            pltpu.SemaphoreType.DMA,
        ],
        compiler_params=pltpu.CompilerParams(use_tc_tiling_on_sc=False),
    )
    def agg_kernel(h_hbm, src_hbm, dst_hbm, z_hbm, out_hbm,
                   src_v, dst_v, rows_v, table_s, sem):
        cid = lax.axis_index("c")
        sid = lax.axis_index("s")
        wid = sid * 2 + cid
        for z in range(ZCH):
            r0 = sid * TROWS + z * CHUNK
            pltpu.sync_copy(z_hbm.at[pl.ds(r0, CHUNK)], rows_v)
            pltpu.sync_copy(rows_v, table_s.at[pl.ds(r0, CHUNK)])
        pltpu.sync_copy(src_hbm.at[wid], src_v)
        pltpu.sync_copy(dst_hbm.at[wid], dst_v)
        plsc.subcore_barrier()

        def body(j, carry):
            pltpu.async_copy(h_hbm.at[src_v.at[j]], rows_v, sem).wait()
            pltpu.sync_copy(rows_v, table_s.at[dst_v.at[j]], add=True)
            return carry

        lax.fori_loop(0, NCH, body, 0)
        plsc.subcore_barrier()
        r0 = sid * TROWS
        pltpu.sync_copy(table_s.at[pl.ds(r0, TROWS)],
                        out_hbm.at[cid, pl.ds(r0, TROWS)])

    return agg_kernel(h, src, dst, zeros_tab)


def _embed(x, W_in, b_in):
    def body(x_ref, w_ref, b_ref, o_ref):
        v = jnp.dot(x_ref[...], w_ref[...],
                    preferred_element_type=jnp.float32) + b_ref[...]
        o_ref[...] = jnp.maximum(v * INV_BN, 0.0)

    return pl.pallas_call(
        body,
        grid=(N // RB,),
        in_specs=[
            pl.BlockSpec((RB, F), lambda i: (i, 0)),
            pl.BlockSpec((F, H), lambda i: (0, 0)),
            pl.BlockSpec((1, H), lambda i: (0, 0)),
        ],
        out_specs=pl.BlockSpec((RB, H), lambda i: (i, 0)),
        out_shape=jax.ShapeDtypeStruct((N, H), jnp.float32),
    )(x, W_in, b_in.reshape(1, H))


def _gin_mlp(h, p, eps, Wa, ba, Wb, bb):
    def body(h_ref, p_ref, e_ref, wa_ref, ba_ref, wb_ref, bb_ref, o_ref):
        a = p_ref[...]
        t = h_ref[...] * (1.0 + e_ref[0, 0]) + a[0] + a[1]
        u = jnp.dot(t, wa_ref[...],
                    preferred_element_type=jnp.float32) + ba_ref[...]
        u = jnp.maximum(u * INV_BN, 0.0)
        v = jnp.dot(u, wb_ref[...],
                    preferred_element_type=jnp.float32) + bb_ref[...]
        o_ref[...] = v * INV_BN + h_ref[...]

    return pl.pallas_call(
        body,
        grid=(N // RB,),
        in_specs=[
            pl.BlockSpec((RB, H), lambda i: (i, 0)),
            pl.BlockSpec((2, RB, H), lambda i: (0, i, 0)),
            pl.BlockSpec((1, 1), lambda i: (0, 0)),
            pl.BlockSpec((H, H), lambda i: (0, 0)),
            pl.BlockSpec((1, H), lambda i: (0, 0)),
            pl.BlockSpec((H, H), lambda i: (0, 0)),
            pl.BlockSpec((1, H), lambda i: (0, 0)),
        ],
        out_specs=pl.BlockSpec((RB, H), lambda i: (i, 0)),
        out_shape=jax.ShapeDtypeStruct((N, H), jnp.float32),
    )(h, p, eps.reshape(1, 1), Wa, ba.reshape(1, H), Wb, bb.reshape(1, H))


def _gin_final(h, p, eps, Wa, ba, Wb, bb, Wc1, bc1, Wc2, bc2):
    G = N // RB

    def body(h_ref, p_ref, e_ref, wa_ref, ba_ref, wb_ref, bb_ref,
             wc1_ref, bc1_ref, wc2_ref, bc2_ref, o_ref, acc_ref):
        i = pl.program_id(0)
        a = p_ref[...]
        t = h_ref[...] * (1.0 + e_ref[0, 0]) + a[0] + a[1]
        u = jnp.dot(t, wa_ref[...],
                    preferred_element_type=jnp.float32) + ba_ref[...]
        u = jnp.maximum(u * INV_BN, 0.0)
        v = jnp.dot(u, wb_ref[...],
                    preferred_element_type=jnp.float32) + bb_ref[...]
        v = v * INV_BN + h_ref[...]
        s = jnp.sum(v, axis=0, keepdims=True)

        @pl.when(i == 0)
        def _():
            acc_ref[...] = s

        @pl.when(i > 0)
        def _():
            acc_ref[...] = acc_ref[...] + s

        @pl.when(i == G - 1)
        def _():
            pooled = acc_ref[...] * (1.0 / N)
            c1 = jnp.dot(pooled, wc1_ref[...],
                         preferred_element_type=jnp.float32) + bc1_ref[...]
            c1 = jnp.maximum(c1 * INV_BN, 0.0)
            o_ref[...] = jnp.dot(c1, wc2_ref[...],
                                 preferred_element_type=jnp.float32) + bc2_ref[...]

    return pl.pallas_call(
        body,
        grid=(G,),
        in_specs=[
            pl.BlockSpec((RB, H), lambda i: (i, 0)),
            pl.BlockSpec((2, RB, H), lambda i: (0, i, 0)),
            pl.BlockSpec((1, 1), lambda i: (0, 0)),
            pl.BlockSpec((H, H), lambda i: (0, 0)),
            pl.BlockSpec((1, H), lambda i: (0, 0)),
            pl.BlockSpec((H, H), lambda i: (0, 0)),
            pl.BlockSpec((1, H), lambda i: (0, 0)),
            pl.BlockSpec((H, H // 2), lambda i: (0, 0)),
            pl.BlockSpec((1, H // 2), lambda i: (0, 0)),
            pl.BlockSpec((H // 2, C), lambda i: (0, 0)),
            pl.BlockSpec((1, C), lambda i: (0, 0)),
        ],
        out_specs=pl.BlockSpec((1, C), lambda i: (0, 0)),
        out_shape=jax.ShapeDtypeStruct((1, C), jnp.float32),
        scratch_shapes=[pltpu.VMEM((1, H), jnp.float32)],
    )(h, p, eps.reshape(1, 1), Wa, ba.reshape(1, H), Wb, bb.reshape(1, H),
      Wc1, bc1.reshape(1, H // 2), Wc2, bc2.reshape(1, C))


def kernel(x, edge_index, W_in, b_in,
           eps1, W1a, b1a, W1b, b1b,
           eps2, W2a, b2a, W2b, b2b,
           eps3, W3a, b3a, W3b, b3b,
           Wc1, bc1, Wc2, bc2):
    ei = edge_index.astype(jnp.int32)
    pad = EPAD - E
    src = jnp.concatenate([ei[0], jnp.zeros((pad,), jnp.int32)])
    dst = jnp.concatenate([ei[1], jnp.full((pad,), N, jnp.int32)])
    src = src.reshape(NW, NCH, CHUNK)
    dst = dst.reshape(NW, NCH, CHUNK)
    zeros_tab = jnp.zeros((NPAD, H), jnp.float32)

    h = _embed(x, W_in, b_in)
    p = _sc_agg(h, src, dst, zeros_tab)
    h = _gin_mlp(h, p, eps1, W1a, b1a, W1b, b1b)
    p = _sc_agg(h, src, dst, zeros_tab)
    h = _gin_mlp(h, p, eps2, W2a, b2a, W2b, b2b)
    p = _sc_agg(h, src, dst, zeros_tab)
    return _gin_final(h, p, eps3, W3a, b3a, W3b, b3b, Wc1, bc1, Wc2, bc2)
